# baseline (speedup 1.0000x reference)
import jax
import jax.numpy as jnp
from jax import lax
from jax.experimental import pallas as pl

H, W = 64, 64

import numpy as _np

_DEINT_NP = _np.zeros((32, 32), dtype=_np.float32)
for _p in range(16):
    _DEINT_NP[2 * _p, _p] = 1.0
    _DEINT_NP[2 * _p + 1, 16 + _p] = 1.0


def _raster_body(x_ref, res_ref, org_ref, deint_ref, out_ref):
    b = pl.program_id(0)
    s = pl.program_id(1)
    P = 16
    xv = x_ref[b, s].reshape(1, 2 * P)
    lane = lax.broadcasted_iota(jnp.int32, (1, 2 * P), 1)
    is_y = (lane % 2) == 1
    denom = jnp.where(is_y, res_ref[b, s, 0], res_ref[b, s, 1])
    shift = jnp.where(is_y, org_ref[b, s, 0], org_ref[b, s, 1])
    idx = (xv / denom + shift).astype(jnp.int32)
    inb = (idx >= 0) & (idx < H)
    idx = jnp.clip(idx, -1, H)
    a = jnp.concatenate(
        [idx.astype(jnp.float32), inb.astype(jnp.float32)], axis=0)
    d = a @ deint_ref[...]
    colv = d[0:1, 0:P].astype(jnp.int32)
    rowv = d[0:1, P:2 * P].astype(jnp.int32)
    valid = (d[1:2, 0:P] + d[1:2, P:2 * P]) == 2.0
    hh = lax.broadcasted_iota(jnp.int32, (H, 1, P), 0)
    ww = lax.broadcasted_iota(jnp.int32, (1, W, P), 1)
    rowm = hh == rowv.reshape(1, 1, P)
    colm = (ww == colv.reshape(1, 1, P)) & valid.reshape(1, 1, P)
    out_ref[0, 0] = (rowm & colm).astype(jnp.float32)


def kernel(x, resolution, origin):
    B, S, n2 = x.shape
    P = n2 // 2
    out = pl.pallas_call(
        _raster_body,
        grid=(B, S),
        in_specs=[
            pl.BlockSpec((B, S, n2), lambda b, s: (0, 0, 0)),
            pl.BlockSpec((B, S, 2), lambda b, s: (0, 0, 0)),
            pl.BlockSpec((B, S, 2), lambda b, s: (0, 0, 0)),
            pl.BlockSpec((32, 32), lambda b, s: (0, 0)),
        ],
        out_specs=pl.BlockSpec((1, 1, H, W, P), lambda b, s: (b, s, 0, 0, 0)),
        out_shape=jax.ShapeDtypeStruct((B, S, H, W, P), jnp.float32),
    )(x, resolution, origin, _DEINT_NP)
    return out


# speedup vs baseline: 1.1400x; 1.1400x over previous
import jax
import jax.numpy as jnp
from jax import lax
from jax.experimental import pallas as pl

H, W = 64, 64

import numpy as _np

_DEINT_NP = _np.zeros((32, 32), dtype=_np.float32)
for _p in range(16):
    _DEINT_NP[2 * _p, _p] = 1.0
    _DEINT_NP[2 * _p + 1, 16 + _p] = 1.0


_SG = 4


def _raster_body(x_ref, res_ref, org_ref, deint_ref, out_ref):
    b = pl.program_id(0)
    s4 = pl.program_id(1)
    P = 16
    for j in range(_SG):
        s = s4 * _SG + j
        xv = x_ref[b, s].reshape(1, 2 * P)
        lane = lax.broadcasted_iota(jnp.int32, (1, 2 * P), 1)
        is_y = (lane % 2) == 1
        denom = jnp.where(is_y, res_ref[b, s, 0], res_ref[b, s, 1])
        shift = jnp.where(is_y, org_ref[b, s, 0], org_ref[b, s, 1])
        idx = (xv / denom + shift).astype(jnp.int32)
        inb = (idx >= 0) & (idx < H)
        idx = jnp.clip(idx, -1, H)
        a = jnp.concatenate(
            [idx.astype(jnp.float32), inb.astype(jnp.float32)], axis=0)
        d = a @ deint_ref[...]
        colv = d[0:1, 0:P].astype(jnp.int32)
        rowv = d[0:1, P:2 * P].astype(jnp.int32)
        valid = (d[1:2, 0:P] + d[1:2, P:2 * P]) == 2.0
        hh = lax.broadcasted_iota(jnp.int32, (H, 1, P), 0)
        ww = lax.broadcasted_iota(jnp.int32, (1, W, P), 1)
        rowm = hh == rowv.reshape(1, 1, P)
        colm = (ww == colv.reshape(1, 1, P)) & valid.reshape(1, 1, P)
        out_ref[0, j] = (rowm & colm).astype(jnp.float32)


def kernel(x, resolution, origin):
    B, S, n2 = x.shape
    P = n2 // 2
    out = pl.pallas_call(
        _raster_body,
        grid=(B, S // _SG),
        in_specs=[
            pl.BlockSpec((B, S, n2), lambda b, s: (0, 0, 0)),
            pl.BlockSpec((B, S, 2), lambda b, s: (0, 0, 0)),
            pl.BlockSpec((B, S, 2), lambda b, s: (0, 0, 0)),
            pl.BlockSpec((32, 32), lambda b, s: (0, 0)),
        ],
        out_specs=pl.BlockSpec((1, _SG, H, W, P), lambda b, s: (b, s, 0, 0, 0)),
        out_shape=jax.ShapeDtypeStruct((B, S, H, W, P), jnp.float32),
    )(x, resolution, origin, _DEINT_NP)
    return out


# speedup vs baseline: 8.9043x; 7.8107x over previous
import jax
import jax.numpy as jnp
from jax import lax
from jax.experimental import pallas as pl

H, W = 64, 64

import numpy as _np

_DEINT_NP = _np.zeros((32, 32), dtype=_np.float32)
for _p in range(16):
    _DEINT_NP[2 * _p, _p] = 1.0
    _DEINT_NP[2 * _p + 1, 16 + _p] = 1.0


_SG = 4


def _raster_body(x_ref, res_ref, org_ref, deint_ref, out_ref):
    b = pl.program_id(0)
    s4 = pl.program_id(1)
    P = 16
    for j in range(_SG):
        s = s4 * _SG + j
        xv = x_ref[b, s].reshape(1, 2 * P)
        lane = lax.broadcasted_iota(jnp.int32, (1, 2 * P), 1)
        is_y = (lane % 2) == 1
        denom = jnp.where(is_y, res_ref[b, s, 0], res_ref[b, s, 1])
        shift = jnp.where(is_y, org_ref[b, s, 0], org_ref[b, s, 1])
        idx = (xv / denom + shift).astype(jnp.int32)
        inb = (idx >= 0) & (idx < H)
        idx = jnp.clip(idx, -1, H)
        a = jnp.concatenate(
            [idx.astype(jnp.float32), inb.astype(jnp.float32)], axis=0)
        dS = lax.dot_general(
            deint_ref[...], a,
            dimension_numbers=(((0,), (1,)), ((), ())),
            preferred_element_type=jnp.float32)
        colv = dS[0:P, 0:1].astype(jnp.int32)
        rowv = dS[P:2 * P, 0:1].astype(jnp.int32)
        valid = (dS[0:P, 1:2] + dS[P:2 * P, 1:2]) == 2.0
        ww = lax.broadcasted_iota(jnp.int32, (P, W), 1)
        colm = (ww == colv) & valid
        rowb = jnp.broadcast_to(rowv, (P, W))
        hh3 = lax.broadcasted_iota(jnp.int32, (H, P, W), 0)
        out3 = (hh3 == rowb[None, :, :]) & colm[None, :, :]
        out_ref[0, j] = out3.astype(jnp.float32)


def kernel(x, resolution, origin):
    B, S, n2 = x.shape
    P = n2 // 2
    out = pl.pallas_call(
        _raster_body,
        grid=(B, S // _SG),
        in_specs=[
            pl.BlockSpec((B, S, n2), lambda b, s: (0, 0, 0)),
            pl.BlockSpec((B, S, 2), lambda b, s: (0, 0, 0)),
            pl.BlockSpec((B, S, 2), lambda b, s: (0, 0, 0)),
            pl.BlockSpec((32, 32), lambda b, s: (0, 0)),
        ],
        out_specs=pl.BlockSpec((1, _SG, H, P, W), lambda b, s: (b, s, 0, 0, 0)),
        out_shape=jax.ShapeDtypeStruct((B, S, H, P, W), jnp.float32),
    )(x, resolution, origin, _DEINT_NP)
    return out.transpose(0, 1, 2, 4, 3)


# speedup vs baseline: 10.5269x; 1.1822x over previous
import jax
import jax.numpy as jnp
from jax import lax
from jax.experimental import pallas as pl

H, W = 64, 64

import numpy as _np

_DEINT_NP = _np.zeros((32, 32), dtype=_np.float32)
for _p in range(16):
    _DEINT_NP[2 * _p, _p] = 1.0
    _DEINT_NP[2 * _p + 1, 16 + _p] = 1.0


_SG = 10


def _raster_body(x_ref, res_ref, org_ref, deint_ref, out_ref):
    b = pl.program_id(0)
    s4 = pl.program_id(1)
    P = 16
    for j in range(_SG):
        s = s4 * _SG + j
        xv = x_ref[b, s].reshape(1, 2 * P)
        lane = lax.broadcasted_iota(jnp.int32, (1, 2 * P), 1)
        is_y = (lane % 2) == 1
        denom = jnp.where(is_y, res_ref[b, s, 0], res_ref[b, s, 1])
        shift = jnp.where(is_y, org_ref[b, s, 0], org_ref[b, s, 1])
        idx = (xv / denom + shift).astype(jnp.int32)
        inb = (idx >= 0) & (idx < H)
        idx = jnp.clip(idx, -1, H)
        a = jnp.concatenate(
            [idx.astype(jnp.float32), inb.astype(jnp.float32)], axis=0)
        dS = lax.dot_general(
            deint_ref[...], a,
            dimension_numbers=(((0,), (1,)), ((), ())),
            preferred_element_type=jnp.float32)
        colv = dS[0:P, 0:1].astype(jnp.int32)
        rowv = dS[P:2 * P, 0:1].astype(jnp.int32)
        valid = (dS[0:P, 1:2] + dS[P:2 * P, 1:2]) == 2.0
        ww = lax.broadcasted_iota(jnp.int32, (P, W), 1)
        colm = (ww == colv) & valid
        rowb = jnp.broadcast_to(rowv, (P, W))
        hh3 = lax.broadcasted_iota(jnp.int32, (H, P, W), 0)
        out3 = (hh3 == rowb[None, :, :]) & colm[None, :, :]
        out_ref[0, j] = out3.astype(jnp.float32)


def kernel(x, resolution, origin):
    B, S, n2 = x.shape
    P = n2 // 2
    out = pl.pallas_call(
        _raster_body,
        grid=(B, S // _SG),
        in_specs=[
            pl.BlockSpec((B, S, n2), lambda b, s: (0, 0, 0)),
            pl.BlockSpec((B, S, 2), lambda b, s: (0, 0, 0)),
            pl.BlockSpec((B, S, 2), lambda b, s: (0, 0, 0)),
            pl.BlockSpec((32, 32), lambda b, s: (0, 0)),
        ],
        out_specs=pl.BlockSpec((1, _SG, H, P, W), lambda b, s: (b, s, 0, 0, 0)),
        out_shape=jax.ShapeDtypeStruct((B, S, H, P, W), jnp.float32),
    )(x, resolution, origin, _DEINT_NP)
    return out.transpose(0, 1, 2, 4, 3)


# speedup vs baseline: 10.7552x; 1.0217x over previous
import jax
import jax.numpy as jnp
from jax import lax
from jax.experimental import pallas as pl

H, W = 64, 64

import numpy as _np

_DEINT_NP = _np.zeros((32, 32), dtype=_np.float32)
for _p in range(16):
    _DEINT_NP[2 * _p, _p] = 1.0
    _DEINT_NP[2 * _p + 1, 16 + _p] = 1.0


_SG = 20


def _raster_body(x_ref, res_ref, org_ref, deint_ref, out_ref):
    b = pl.program_id(0)
    s4 = pl.program_id(1)
    P = 16
    for j in range(_SG):
        s = s4 * _SG + j
        xv = x_ref[b, s].reshape(1, 2 * P)
        lane = lax.broadcasted_iota(jnp.int32, (1, 2 * P), 1)
        is_y = (lane % 2) == 1
        denom = jnp.where(is_y, res_ref[b, s, 0], res_ref[b, s, 1])
        shift = jnp.where(is_y, org_ref[b, s, 0], org_ref[b, s, 1])
        idx = (xv / denom + shift).astype(jnp.int32)
        inb = (idx >= 0) & (idx < H)
        idx = jnp.clip(idx, -1, H)
        a = jnp.concatenate(
            [idx.astype(jnp.float32), inb.astype(jnp.float32)], axis=0)
        dS = lax.dot_general(
            deint_ref[...], a,
            dimension_numbers=(((0,), (1,)), ((), ())),
            preferred_element_type=jnp.float32)
        colv = dS[0:P, 0:1].astype(jnp.int32)
        rowv = dS[P:2 * P, 0:1].astype(jnp.int32)
        valid = (dS[0:P, 1:2] + dS[P:2 * P, 1:2]) == 2.0
        ww = lax.broadcasted_iota(jnp.int32, (P, W), 1)
        colm = (ww == colv) & valid
        rowb = jnp.broadcast_to(rowv, (P, W))
        hh3 = lax.broadcasted_iota(jnp.int32, (H, P, W), 0)
        out3 = (hh3 == rowb[None, :, :]) & colm[None, :, :]
        out_ref[0, j] = out3.astype(jnp.float32)


def kernel(x, resolution, origin):
    B, S, n2 = x.shape
    P = n2 // 2
    out = pl.pallas_call(
        _raster_body,
        grid=(B, S // _SG),
        in_specs=[
            pl.BlockSpec((B, S, n2), lambda b, s: (0, 0, 0)),
            pl.BlockSpec((B, S, 2), lambda b, s: (0, 0, 0)),
            pl.BlockSpec((B, S, 2), lambda b, s: (0, 0, 0)),
            pl.BlockSpec((32, 32), lambda b, s: (0, 0)),
        ],
        out_specs=pl.BlockSpec((1, _SG, H, P, W), lambda b, s: (b, s, 0, 0, 0)),
        out_shape=jax.ShapeDtypeStruct((B, S, H, P, W), jnp.float32),
    )(x, resolution, origin, _DEINT_NP)
    return out.transpose(0, 1, 2, 4, 3)


# speedup vs baseline: 12.4139x; 1.1542x over previous
import jax
import jax.numpy as jnp
from jax import lax
from jax.experimental import pallas as pl

H, W = 64, 64

import numpy as _np

_DEINT_NP = _np.zeros((32, 32), dtype=_np.float32)
for _p in range(16):
    _DEINT_NP[2 * _p, _p] = 1.0
    _DEINT_NP[2 * _p + 1, 16 + _p] = 1.0


_SG = 20


def _raster_body(x_ref, res_ref, org_ref, deint_ref, out_ref):
    b = pl.program_id(0)
    s4 = pl.program_id(1)
    P = 16
    for j in range(_SG):
        s = s4 * _SG + j
        xv = x_ref[b, s].reshape(1, 2 * P)
        lane = lax.broadcasted_iota(jnp.int32, (1, 2 * P), 1)
        is_y = (lane % 2) == 1
        denom = jnp.where(is_y, res_ref[b, s, 0], res_ref[b, s, 1])
        shift = jnp.where(is_y, org_ref[b, s, 0], org_ref[b, s, 1])
        idx = (xv / denom + shift).astype(jnp.int32)
        inb = (idx >= 0) & (idx < H)
        idx = jnp.clip(idx, -1, H)
        a = jnp.concatenate(
            [idx.astype(jnp.float32), inb.astype(jnp.float32)], axis=0)
        dS = lax.dot_general(
            deint_ref[...], a,
            dimension_numbers=(((0,), (1,)), ((), ())),
            preferred_element_type=jnp.float32)
        colv = dS[0:P, 0:1].astype(jnp.int32)
        rowv = dS[P:2 * P, 0:1].astype(jnp.int32)
        valid = (dS[0:P, 1:2] + dS[P:2 * P, 1:2]) == 2.0
        ww = lax.broadcasted_iota(jnp.int32, (P, W), 1)
        colm = (ww == colv) & valid
        colmf = colm.astype(jnp.float32)
        rowb = jnp.broadcast_to(rowv, (P, W))
        hh3 = lax.broadcasted_iota(jnp.int32, (H, P, W), 0)
        out_ref[0, j] = jnp.where(hh3 == rowb[None, :, :], colmf[None, :, :],
                                  0.0)


def kernel(x, resolution, origin):
    B, S, n2 = x.shape
    P = n2 // 2
    out = pl.pallas_call(
        _raster_body,
        grid=(B, S // _SG),
        in_specs=[
            pl.BlockSpec((B, S, n2), lambda b, s: (0, 0, 0)),
            pl.BlockSpec((B, S, 2), lambda b, s: (0, 0, 0)),
            pl.BlockSpec((B, S, 2), lambda b, s: (0, 0, 0)),
            pl.BlockSpec((32, 32), lambda b, s: (0, 0)),
        ],
        out_specs=pl.BlockSpec((1, _SG, H, P, W), lambda b, s: (b, s, 0, 0, 0)),
        out_shape=jax.ShapeDtypeStruct((B, S, H, P, W), jnp.float32),
    )(x, resolution, origin, _DEINT_NP)
    return out.transpose(0, 1, 2, 4, 3)
